# baseline (speedup 1.0000x reference)
import functools

import jax
import jax.numpy as jnp
from jax import lax
from jax.experimental import pallas as pl
from jax.experimental.pallas import tpu as pltpu
from jax.experimental.pallas import tpu_sc as plsc

N_USERS = 1000000
N_ITEMS = 100000
D = 64
B = 16384

NC = 2
NS = 16
NW = NC * NS
B_PER_W = B // NW
CH = 16
N_CH = B_PER_W // CH
STREAMS = D * CH // 128


def _mf_kernel(u_hbm, i_hbm, ut_hbm, it_hbm,
               pred_hbm, p_hbm, q_hbm,
               idx_u, idx_i, ixu, ixi, pu_v, qu_v, p_acc, q_acc,
               pred_v, sem_g, sem_o):
    wid = lax.axis_index("s") * NC + lax.axis_index("c")
    base = wid * B_PER_W

    pltpu.sync_copy(u_hbm.at[pl.ds(base, B_PER_W)], idx_u)
    pltpu.sync_copy(i_hbm.at[pl.ds(base, B_PER_W)], idx_i)

    def body(c, carry):
        u16 = idx_u[pl.ds(c * CH, CH)]
        i16 = idx_i[pl.ds(c * CH, CH)]
        for d in range(D):
            ixu[d // 8, pl.ds((d % 8) * CH, CH)] = u16 + d * N_USERS
            ixi[d // 8, pl.ds((d % 8) * CH, CH)] = i16 + d * N_ITEMS
        gathers = []
        for j in range(STREAMS):
            gathers.append(pltpu.async_copy(
                ut_hbm.at[ixu.at[j]], pu_v.at[pl.ds(j * 128, 128)], sem_g))
            gathers.append(pltpu.async_copy(
                it_hbm.at[ixi.at[j]], qu_v.at[pl.ds(j * 128, 128)], sem_g))
        for g_ in gathers:
            g_.wait()

        acc = None
        for d in range(D):
            pv = pu_v[pl.ds(d * CH, CH)]
            qv = qu_v[pl.ds(d * CH, CH)]
            p_acc[d, pl.ds(c * CH, CH)] = pv
            q_acc[d, pl.ds(c * CH, CH)] = qv
            acc = pv * qv if acc is None else acc + pv * qv
        pred_v[pl.ds(c * CH, CH)] = acc
        return carry

    lax.fori_loop(0, N_CH, body, 0)

    pltpu.sync_copy(p_acc, p_hbm.at[:, pl.ds(base, B_PER_W)])
    pltpu.sync_copy(q_acc, q_hbm.at[:, pl.ds(base, B_PER_W)])
    pltpu.sync_copy(pred_v, pred_hbm.at[pl.ds(base, B_PER_W)])


@jax.jit
def _mf(u, i, user_table, item_table):
    mesh = plsc.VectorSubcoreMesh(core_axis_name="c", subcore_axis_name="s")
    run = functools.partial(
        pl.kernel,
        out_type=(
            jax.ShapeDtypeStruct((B,), jnp.float32),
            jax.ShapeDtypeStruct((D, B), jnp.float32),
            jax.ShapeDtypeStruct((D, B), jnp.float32),
        ),
        mesh=mesh,
        compiler_params=pltpu.CompilerParams(needs_layout_passes=False),
        scratch_types=[
            pltpu.VMEM((B_PER_W,), jnp.int32),
            pltpu.VMEM((B_PER_W,), jnp.int32),
            pltpu.VMEM((STREAMS, 128), jnp.int32),
            pltpu.VMEM((STREAMS, 128), jnp.int32),
            pltpu.VMEM((D * CH,), jnp.float32),
            pltpu.VMEM((D * CH,), jnp.float32),
            pltpu.VMEM((D, B_PER_W), jnp.float32),
            pltpu.VMEM((D, B_PER_W), jnp.float32),
            pltpu.VMEM((B_PER_W,), jnp.float32),
            pltpu.SemaphoreType.DMA,
            pltpu.SemaphoreType.DMA,
        ],
    )(_mf_kernel)
    ut_f = user_table.T.reshape(D * N_USERS)
    it_f = item_table.T.reshape(D * N_ITEMS)
    pred, p_t, q_t = run(u, i, ut_f, it_f)
    return pred, p_t.T.reshape(B, 1, D), q_t.T.reshape(B, D, 1)


def kernel(u, i, user_table, item_table):
    return _mf(u, i, user_table, item_table)


# speedup vs baseline: 7.4848x; 7.4848x over previous
import functools

import jax
import jax.numpy as jnp
from jax import lax
from jax.experimental import pallas as pl
from jax.experimental.pallas import tpu as pltpu
from jax.experimental.pallas import tpu_sc as plsc

N_USERS = 1000000
N_ITEMS = 100000
D = 64
B = 16384

NC = 2
NS = 16
NW = NC * NS
B_PER_W = B // NW
IDX_CHUNK = 128
N_CHUNKS = B_PER_W // IDX_CHUNK


def _mf_kernel(u_hbm, i_hbm, ut_hbm, it_hbm,
               pred_hbm, p_hbm, q_hbm,
               idx_u, idx_i, p_v, q_v, pred_v, sem_u, sem_i):
    wid = lax.axis_index("s") * NC + lax.axis_index("c")
    row_base = wid * N_CHUNKS

    pltpu.sync_copy(u_hbm.at[pl.ds(row_base, N_CHUNKS)], idx_u)
    pltpu.sync_copy(i_hbm.at[pl.ds(row_base, N_CHUNKS)], idx_i)

    copies = []
    for j in range(N_CHUNKS):
        copies.append(pltpu.async_copy(
            ut_hbm.at[idx_u.at[j]], p_v.at[pl.ds(j * IDX_CHUNK, IDX_CHUNK)], sem_u))
        copies.append(pltpu.async_copy(
            it_hbm.at[idx_i.at[j]], q_v.at[pl.ds(j * IDX_CHUNK, IDX_CHUNK)], sem_i))
    for c in copies:
        c.wait()

    lanes = lax.iota(jnp.int32, 16)

    def body(g, carry):
        out = jnp.zeros((16,), jnp.float32)
        for r in range(16):
            b = g * 16 + r
            acc = p_v[b, pl.ds(0, 16)] * q_v[b, pl.ds(0, 16)]
            for c in range(1, D // 16):
                acc = acc + p_v[b, pl.ds(c * 16, 16)] * q_v[b, pl.ds(c * 16, 16)]
            out = jnp.where(lanes == r, jnp.sum(acc), out)
        pred_v[pl.ds(g * 16, 16)] = out
        return carry

    lax.fori_loop(0, B_PER_W // 16, body, 0)

    base = wid * B_PER_W
    pltpu.sync_copy(p_v, p_hbm.at[pl.ds(base, B_PER_W)])
    pltpu.sync_copy(q_v, q_hbm.at[pl.ds(base, B_PER_W)])
    pltpu.sync_copy(pred_v, pred_hbm.at[pl.ds(base, B_PER_W)])


@jax.jit
def _mf(u, i, user_table, item_table):
    mesh = plsc.VectorSubcoreMesh(core_axis_name="c", subcore_axis_name="s")
    run = functools.partial(
        pl.kernel,
        out_type=(
            jax.ShapeDtypeStruct((B,), jnp.float32),
            jax.ShapeDtypeStruct((B, D), jnp.float32),
            jax.ShapeDtypeStruct((B, D), jnp.float32),
        ),
        mesh=mesh,
        compiler_params=pltpu.CompilerParams(
            needs_layout_passes=False, use_tc_tiling_on_sc=False),
        scratch_types=[
            pltpu.VMEM((N_CHUNKS, IDX_CHUNK), jnp.int32),
            pltpu.VMEM((N_CHUNKS, IDX_CHUNK), jnp.int32),
            pltpu.VMEM((B_PER_W, D), jnp.float32),
            pltpu.VMEM((B_PER_W, D), jnp.float32),
            pltpu.VMEM((B_PER_W,), jnp.float32),
            pltpu.SemaphoreType.DMA,
            pltpu.SemaphoreType.DMA,
        ],
    )(_mf_kernel)
    u2 = u.reshape(B // IDX_CHUNK, IDX_CHUNK)
    i2 = i.reshape(B // IDX_CHUNK, IDX_CHUNK)
    pred, p, q = run(u2, i2, user_table, item_table)
    return pred, p.reshape(B, 1, D), q.reshape(B, D, 1)


def kernel(u, i, user_table, item_table):
    return _mf(u, i, user_table, item_table)
